# baseline (speedup 1.0000x reference)
import jax
import jax.numpy as jnp
from jax import lax
from jax.experimental import pallas as pl
from jax.experimental.pallas import tpu as pltpu
from jax.experimental.pallas import tpu_sc as plsc

_CB = 8
_TILES = 32


def kernel(token_ids, param):
    B, S = token_ids.shape
    V, D = param.shape
    rows_per_tile = B // _TILES
    chunks = rows_per_tile // _CB
    gather_bytes = _CB * S * 128 * 4
    write_bytes = _CB * S * D * 4

    padded = jnp.pad(param, ((0, 0), (0, 128 - D)))
    idx = token_ids.astype(jnp.int32)

    mesh = plsc.VectorSubcoreMesh(core_axis_name="c", subcore_axis_name="s")

    @pl.kernel(
        out_type=jax.ShapeDtypeStruct((B, S, D), param.dtype),
        mesh=mesh,
        scratch_types=[
            pltpu.VMEM((_CB, S), jnp.int32),
            pltpu.VMEM((_CB * S, 128), jnp.float32),
            pltpu.VMEM((_CB, S, D), jnp.float32),
            pltpu.SemaphoreType.DMA,
            pltpu.SemaphoreType.DMA,
        ],
    )
    def gather_kernel(table_hbm, idx_hbm, out_hbm, ibuf, rbuf, obuf, gsem, wsem):
        wid = lax.axis_index("s") * 2 + lax.axis_index("c")
        base = wid * rows_per_tile

        @pl.loop(0, chunks)
        def _(c):
            b0 = base + c * _CB
            pltpu.sync_copy(idx_hbm.at[pl.ds(b0, _CB)], ibuf)
            gathers = [
                pltpu.async_copy(
                    table_hbm.at[ibuf.at[j]],
                    rbuf.at[pl.ds(j * S, S)],
                    gsem,
                )
                for j in range(_CB)
            ]
            for h in gathers:
                h.wait()

            @pl.loop(0, S)
            def _(t):
                for j in range(_CB):
                    for h in range(D // 16):
                        obuf[j, t, pl.ds(h * 16, 16)] = rbuf[
                            j * S + t, pl.ds(h * 16, 16)
                        ]

            pltpu.async_copy(
                obuf, out_hbm.at[pl.ds(b0, _CB)], wsem
            ).wait()

    out = gather_kernel(padded, idx)
    return out


# speedup vs baseline: 1.1947x; 1.1947x over previous
import jax
import jax.numpy as jnp
from jax import lax
from jax.experimental import pallas as pl
from jax.experimental.pallas import tpu as pltpu
from jax.experimental.pallas import tpu_sc as plsc

_CB = 8
_TILES = 32


def kernel(token_ids, param):
    B, S = token_ids.shape
    V, D = param.shape
    rows_per_tile = B // _TILES
    pairs = rows_per_tile // (2 * _CB)

    padded = jnp.pad(param, ((0, 0), (0, 128 - D)))
    idx = token_ids.astype(jnp.int32)

    mesh = plsc.VectorSubcoreMesh(core_axis_name="c", subcore_axis_name="s")

    @pl.kernel(
        out_type=jax.ShapeDtypeStruct((B, S, 128), param.dtype),
        mesh=mesh,
        scratch_types=[
            pltpu.VMEM((2 * _CB, S), jnp.int32),
            pltpu.VMEM((2, _CB * S, 128), jnp.float32),
            pltpu.SemaphoreType.DMA,
            pltpu.SemaphoreType.DMA,
        ],
    )
    def gather_kernel(table_hbm, idx_hbm, out_hbm, ibuf, rbuf, gsem, wsem):
        wid = lax.axis_index("s") * 2 + lax.axis_index("c")
        base = wid * rows_per_tile

        def fire_gathers(buf_slot, row0, idx0):
            return [
                pltpu.async_copy(
                    table_hbm.at[ibuf.at[idx0 + j]],
                    rbuf.at[buf_slot, pl.ds(j * S, S)],
                    gsem,
                )
                for j in range(_CB)
            ]

        def fire_writes(buf_slot, row0):
            return [
                pltpu.async_copy(
                    rbuf.at[buf_slot, pl.ds(j * S, S)],
                    out_hbm.at[row0 + j],
                    wsem,
                )
                for j in range(_CB)
            ]

        @pl.loop(0, pairs)
        def _(p):
            b0 = base + p * 2 * _CB
            pltpu.sync_copy(idx_hbm.at[pl.ds(b0, 2 * _CB)], ibuf)
            ga = fire_gathers(0, b0, 0)
            for h in ga:
                h.wait()
            wa = fire_writes(0, b0)
            gb = fire_gathers(1, b0 + _CB, _CB)
            for h in wa:
                h.wait()
            for h in gb:
                h.wait()
            wb = fire_writes(1, b0 + _CB)
            for h in wb:
                h.wait()

    out = gather_kernel(padded, idx)
    return out[..., :D]


# speedup vs baseline: 1.2433x; 1.0406x over previous
import jax
import jax.numpy as jnp
from jax import lax
from jax.experimental import pallas as pl
from jax.experimental.pallas import tpu as pltpu
from jax.experimental.pallas import tpu_sc as plsc

_CB = 8
_TILES = 32


def kernel(token_ids, param):
    B, S = token_ids.shape
    V, D = param.shape
    rows_per_tile = B // _TILES
    pairs = rows_per_tile // (2 * _CB)

    padded = jnp.pad(param, ((0, 0), (0, 128 - D)))
    idx = token_ids.astype(jnp.int32)

    mesh = plsc.VectorSubcoreMesh(core_axis_name="c", subcore_axis_name="s")

    @pl.kernel(
        out_type=jax.ShapeDtypeStruct((B, S, 128), param.dtype),
        mesh=mesh,
        scratch_types=[
            pltpu.VMEM((2 * _CB, S), jnp.int32),
            pltpu.VMEM((2, _CB * S, 128), jnp.float32),
            pltpu.SemaphoreType.DMA,
            pltpu.SemaphoreType.DMA,
        ],
    )
    def gather_kernel(table_hbm, idx_hbm, out_hbm, ibuf, rbuf, gsem, wsem):
        wid = lax.axis_index("s") * 2 + lax.axis_index("c")
        base = wid * rows_per_tile

        def fire_gathers(buf_slot, row0, idx0):
            return [
                pltpu.async_copy(
                    table_hbm.at[ibuf.at[idx0 + j]],
                    rbuf.at[buf_slot, pl.ds(j * S, S)],
                    gsem,
                )
                for j in range(_CB)
            ]

        def fire_writes(buf_slot, row0):
            return [
                pltpu.async_copy(
                    rbuf.at[buf_slot, pl.ds(j * S, S)],
                    out_hbm.at[row0 + j],
                    wsem,
                )
                for j in range(_CB)
            ]

        @pl.loop(0, pairs)
        def _(p):
            b0 = base + p * 2 * _CB
            pltpu.sync_copy(idx_hbm.at[pl.ds(b0, 2 * _CB)], ibuf)
            ga = fire_gathers(0, b0, 0)
            gb = fire_gathers(1, b0 + _CB, _CB)
            for h in ga:
                h.wait()
            wa = fire_writes(0, b0)
            for h in gb:
                h.wait()
            wb = fire_writes(1, b0 + _CB)
            for h in wa:
                h.wait()
            for h in wb:
                h.wait()

    out = gather_kernel(padded, idx)
    return out[..., :D]


# speedup vs baseline: 1.2490x; 1.0046x over previous
import jax
import jax.numpy as jnp
from jax import lax
from jax.experimental import pallas as pl
from jax.experimental.pallas import tpu as pltpu
from jax.experimental.pallas import tpu_sc as plsc

_CB = 8
_TILES = 32


def kernel(token_ids, param):
    B, S = token_ids.shape
    V, D = param.shape
    rows_per_tile = B // _TILES
    pairs = rows_per_tile // (2 * _CB)

    padded = jnp.pad(param, ((0, 0), (0, 128 - D)))
    idx = token_ids.astype(jnp.int32)

    mesh = plsc.VectorSubcoreMesh(core_axis_name="c", subcore_axis_name="s")

    @pl.kernel(
        out_type=jax.ShapeDtypeStruct((B, S, 128), param.dtype),
        mesh=mesh,
        scratch_types=[
            pltpu.VMEM((128, S), jnp.int32),
            pltpu.VMEM((2, _CB * S, 128), jnp.float32),
            pltpu.SemaphoreType.DMA,
            pltpu.SemaphoreType.DMA,
        ],
    )
    def gather_kernel(table_hbm, idx_hbm, out_hbm, ibuf, rbuf, gsem, wsem):
        wid = lax.axis_index("s") * 2 + lax.axis_index("c")
        base = wid * rows_per_tile

        def fire_gathers(buf_slot, row0, idx0):
            return [
                pltpu.async_copy(
                    table_hbm.at[ibuf.at[idx0 + j]],
                    rbuf.at[buf_slot, pl.ds(j * S, S)],
                    gsem,
                )
                for j in range(_CB)
            ]

        def fire_writes(buf_slot, row0):
            return [
                pltpu.async_copy(
                    rbuf.at[buf_slot, pl.ds(j * S, S)],
                    out_hbm.at[row0 + j],
                    wsem,
                )
                for j in range(_CB)
            ]

        @pl.loop(0, rows_per_tile // 128)
        def _(q):
            qb = base + q * 128
            pltpu.sync_copy(idx_hbm.at[pl.ds(qb, 128)], ibuf)

            @pl.loop(0, 128 // (2 * _CB))
            def _(r):
                b0 = qb + r * 2 * _CB
                i0 = r * 2 * _CB
                ga = fire_gathers(0, b0, i0)
                gb = fire_gathers(1, b0 + _CB, i0 + _CB)
                for h in ga:
                    h.wait()
                wa = fire_writes(0, b0)
                for h in gb:
                    h.wait()
                wb = fire_writes(1, b0 + _CB)
                for h in wa:
                    h.wait()
                for h in wb:
                    h.wait()

    out = gather_kernel(padded, idx)
    return out[..., :D]
